# Initial kernel scaffold; baseline (speedup 1.0000x reference)
import jax
import jax.numpy as jnp
from jax.experimental import pallas as pl


def kernel(pred, edges):
    raise NotImplementedError("write your pallas kernel here")


# speedup vs baseline: 7.2256x; 7.2256x over previous
import functools

import jax
import jax.numpy as jnp
from jax import lax
from jax.experimental import pallas as pl
from jax.experimental.pallas import tpu as pltpu
from jax.experimental.pallas import tpu_sc as plsc

L = 16
NC = 2
NS = 16
NW = NC * NS

CHUNK_IDX = 100
EDGES_PER_CHUNK = CHUNK_IDX // 2


@functools.lru_cache(maxsize=None)
def _build(n_points, bd, nchunks):
    assert bd % L == 0
    assert nchunks % 2 == 0

    mesh = plsc.VectorSubcoreMesh(core_axis_name="c", subcore_axis_name="s")

    @functools.partial(
        pl.kernel,
        mesh=mesh,
        compiler_params=pltpu.CompilerParams(use_tc_tiling_on_sc=False),
        out_type=jax.ShapeDtypeStruct((NW * L,), jnp.float32),
        scratch_types=[
            pltpu.VMEM((nchunks, CHUNK_IDX), jnp.int32),
            pltpu.VMEM((CHUNK_IDX, bd), jnp.float32),
            pltpu.VMEM((CHUNK_IDX, bd), jnp.float32),
            pltpu.VMEM((L,), jnp.float32),
            pltpu.SemaphoreType.DMA,
            pltpu.SemaphoreType.DMA,
        ],
    )
    def edge_mse(table_hbm, idx_hbm, out_hbm, idx_v, rows0, rows1, acc_v,
                 sem0, sem1):
        wid = lax.axis_index("s") * NC + lax.axis_index("c")
        pltpu.sync_copy(idx_hbm.at[wid], idx_v)

        rows = (rows0, rows1)
        sems = (sem0, sem1)

        def issue(c, b):
            pltpu.async_copy(table_hbm.at[idx_v.at[c]], rows[b], sems[b])

        def wait(b):
            pltpu.make_async_copy(table_hbm.at[idx_v.at[0]], rows[b],
                                  sems[b]).wait()

        def chunk_sum(rows_ref, acc):
            def edge(j, acc):
                s = 2 * j
                for k in range(bd // L):
                    a = rows_ref[s, pl.ds(L * k, L)]
                    b = rows_ref[s + 1, pl.ds(L * k, L)]
                    d = a - b
                    acc = acc + d * d
                return acc
            return lax.fori_loop(0, EDGES_PER_CHUNK, edge, acc, unroll=2)

        issue(0, 0)
        issue(1, 1)

        def outer(g, acc):
            for b in range(2):
                wait(b)
                acc = chunk_sum(rows[b], acc)
                issue(2 * g + 2 + b, b)
            return acc

        acc = jnp.zeros((L,), jnp.float32)
        acc = lax.fori_loop(0, nchunks // 2 - 1, outer, acc)
        for b in range(2):
            wait(b)
            acc = chunk_sum(rows[b], acc)

        acc_v[...] = acc
        pltpu.sync_copy(acc_v, out_hbm.at[pl.ds(wid * L, L)])

    return edge_mse


def kernel(pred, edges):
    B, N, D = pred.shape
    E = edges.shape[0]
    assert (2 * E) % (NW * CHUNK_IDX) == 0
    nchunks = (2 * E) // (NW * CHUNK_IDX)
    table = jnp.transpose(pred, (1, 0, 2)).reshape(N, B * D)
    idx = edges.reshape(NW, nchunks, CHUNK_IDX)
    partials = _build(N, B * D, nchunks)(table, idx)
    return jnp.sum(partials) / jnp.float32(B * E)
